# Initial kernel scaffold; baseline (speedup 1.0000x reference)
import jax
import jax.numpy as jnp
from jax.experimental import pallas as pl


def kernel(node_attr, edge_attr, edge_index, num_nodes, eb_W1, eb_b1, eb_W2, eb_b2, eb_W3, eb_b3, eb_W4, eb_b4, eb_g, eb_beta, nb_W1, nb_b1, nb_W2, nb_b2, nb_W3, nb_b3, nb_W4, nb_b4, nb_g, nb_beta):
    raise NotImplementedError("write your pallas kernel here")


# speedup vs baseline: 3.4025x; 3.4025x over previous
import functools

import jax
import jax.numpy as jnp
from jax import lax
from jax.experimental import pallas as pl
from jax.experimental.pallas import tpu as pltpu
from jax.experimental.pallas import tpu_sc as plsc

H = 128
CHUNK = 128


def _sc_mesh():
    return plsc.VectorSubcoreMesh(
        core_axis_name="c", subcore_axis_name="s", num_cores=2, num_subcores=16
    )


def _make_gather(N, E):
    rows = E // CHUNK
    nworker = 32
    base = rows // nworker
    extra = rows % nworker

    mesh = _sc_mesh()

    @functools.partial(
        pl.kernel,
        out_type=(
            jax.ShapeDtypeStruct((E, H), jnp.float32),
            jax.ShapeDtypeStruct((E, H), jnp.float32),
        ),
        mesh=mesh,
        scratch_types=[
            pltpu.VMEM((base + 1, 1, CHUNK), jnp.int32),
            pltpu.VMEM((base + 1, 1, CHUNK), jnp.int32),
            pltpu.VMEM((CHUNK, H), jnp.float32),
            pltpu.VMEM((CHUNK, H), jnp.float32),
            pltpu.SemaphoreType.DMA,
            pltpu.SemaphoreType.DMA,
        ],
    )
    def gather_k(node_hbm, sidx_hbm, ridx_hbm, sa_hbm, ra_hbm,
                 sidx_v, ridx_v, bufs, bufr, sems, semr):
        cid = lax.axis_index("c")
        sid = lax.axis_index("s")
        wid = sid * 2 + cid
        start = wid * base + jnp.minimum(wid, extra)
        has_extra = wid < extra

        pltpu.sync_copy(sidx_hbm.at[pl.ds(start, base)], sidx_v.at[pl.ds(0, base)])
        pltpu.sync_copy(ridx_hbm.at[pl.ds(start, base)], ridx_v.at[pl.ds(0, base)])

        @pl.when(has_extra)
        def _():
            pltpu.sync_copy(sidx_hbm.at[pl.ds(start + base, 1)],
                            sidx_v.at[pl.ds(base, 1)])
            pltpu.sync_copy(ridx_hbm.at[pl.ds(start + base, 1)],
                            ridx_v.at[pl.ds(base, 1)])

        def process(j):
            ebase = (start + j) * CHUNK
            d1 = pltpu.async_copy(node_hbm.at[sidx_v.at[j, 0]], bufs, sems)
            d2 = pltpu.async_copy(node_hbm.at[ridx_v.at[j, 0]], bufr, semr)
            d1.wait()
            pltpu.sync_copy(bufs, sa_hbm.at[pl.ds(ebase, CHUNK)])
            d2.wait()
            pltpu.sync_copy(bufr, ra_hbm.at[pl.ds(ebase, CHUNK)])

        def body(j, _):
            process(j)
            return _

        lax.fori_loop(0, base, body, None)

        @pl.when(has_extra)
        def _():
            process(base)

    return gather_k


def _make_scatter(N, E):
    rows = E // CHUNK
    rows_core = rows // 2
    base = rows_core // 16
    extra = rows_core % 16
    npart = (N // 16) // 8 * 8
    ntail = N - 16 * npart

    mesh = _sc_mesh()

    @functools.partial(
        pl.kernel,
        out_type=jax.ShapeDtypeStruct((2, N, H), jnp.float32),
        mesh=mesh,
        scratch_types=[
            pltpu.VMEM((base + 1, 1, CHUNK), jnp.int32),
            pltpu.VMEM((CHUNK, H), jnp.float32),
            pltpu.VMEM_SHARED((N, H), jnp.float32),
        ],
    )
    def scatter_k(y_hbm, ridx_hbm, zeros_hbm, agg_hbm, ridx_v, ybuf, acc_sh):
        cid = lax.axis_index("c")
        sid = lax.axis_index("s")

        pltpu.sync_copy(zeros_hbm.at[pl.ds(sid * npart, npart)],
                        acc_sh.at[pl.ds(sid * npart, npart)])
        if ntail:
            @pl.when(sid == 15)
            def _():
                pltpu.sync_copy(zeros_hbm.at[pl.ds(16 * npart, ntail)],
                                acc_sh.at[pl.ds(16 * npart, ntail)])
        plsc.subcore_barrier()

        lstart = sid * base + jnp.minimum(sid, extra)
        grow = cid * rows_core + lstart
        has_extra = sid < extra

        pltpu.sync_copy(ridx_hbm.at[pl.ds(grow, base)], ridx_v.at[pl.ds(0, base)])

        @pl.when(has_extra)
        def _():
            pltpu.sync_copy(ridx_hbm.at[pl.ds(grow + base, 1)],
                            ridx_v.at[pl.ds(base, 1)])

        def process(j):
            ebase = (grow + j) * CHUNK
            pltpu.sync_copy(y_hbm.at[pl.ds(ebase, CHUNK)], ybuf)
            pltpu.sync_copy(ybuf, acc_sh.at[ridx_v.at[j, 0]], add=True)

        def body(j, _):
            process(j)
            return _

        lax.fori_loop(0, base, body, None)

        @pl.when(has_extra)
        def _():
            process(base)

        plsc.subcore_barrier()
        pltpu.sync_copy(acc_sh.at[pl.ds(sid * npart, npart)],
                        agg_hbm.at[cid, pl.ds(sid * npart, npart)])
        if ntail:
            @pl.when(sid == 15)
            def _():
                pltpu.sync_copy(acc_sh.at[pl.ds(16 * npart, ntail)],
                                agg_hbm.at[cid, pl.ds(16 * npart, ntail)])

    return scatter_k


def _ln(h, g, beta):
    mu = jnp.mean(h, axis=-1, keepdims=True)
    d = h - mu
    var = jnp.mean(d * d, axis=-1, keepdims=True)
    return d / jnp.sqrt(var + 1e-5) * g + beta


def _edge_body(sa_ref, ra_ref, ea_ref, w1a, w1b, w1c, b1, w2, b2, w3, b3,
               w4, b4, g, beta, y_ref, ye_ref):
    s = sa_ref[...]
    r = ra_ref[...]
    e = ea_ref[...]
    f32 = jnp.float32
    h = jnp.dot(s, w1a[...], preferred_element_type=f32)
    h = h + jnp.dot(r, w1b[...], preferred_element_type=f32)
    h = h + jnp.dot(e, w1c[...], preferred_element_type=f32)
    h = jax.nn.relu(h + b1[...])
    h = jax.nn.relu(jnp.dot(h, w2[...], preferred_element_type=f32) + b2[...])
    h = jax.nn.relu(jnp.dot(h, w3[...], preferred_element_type=f32) + b3[...])
    h = jnp.dot(h, w4[...], preferred_element_type=f32) + b4[...]
    y = _ln(h, g[...], beta[...])
    y_ref[...] = y
    ye_ref[...] = y + e


def _node_body(na_ref, agg_ref, w1a, w1b, b1, w2, b2, w3, b3, w4, b4, g, beta,
               out_ref):
    n = na_ref[...]
    a = agg_ref[0] + agg_ref[1]
    f32 = jnp.float32
    h = jnp.dot(n, w1a[...], preferred_element_type=f32)
    h = h + jnp.dot(a, w1b[...], preferred_element_type=f32)
    h = jax.nn.relu(h + b1[...])
    h = jax.nn.relu(jnp.dot(h, w2[...], preferred_element_type=f32) + b2[...])
    h = jax.nn.relu(jnp.dot(h, w3[...], preferred_element_type=f32) + b3[...])
    h = jnp.dot(h, w4[...], preferred_element_type=f32) + b4[...]
    out_ref[...] = _ln(h, g[...], beta[...]) + n


def _full(shape):
    nd = len(shape)
    return pl.BlockSpec(shape, lambda i: (0,) * nd)


def _edge_mlp(sa, ra, ea, w1a, w1b, w1c, b1, w2, b2, w3, b3, w4, b4, g, beta,
              block):
    E = sa.shape[0]
    grid = E // block
    row = pl.BlockSpec((block, H), lambda i: (i, 0))
    wspec = _full((H, H))
    vspec = _full((1, H))
    return pl.pallas_call(
        _edge_body,
        grid=(grid,),
        in_specs=[row, row, row, wspec, wspec, wspec, vspec, wspec, vspec,
                  wspec, vspec, wspec, vspec, vspec, vspec],
        out_specs=[row, row],
        out_shape=(
            jax.ShapeDtypeStruct((E, H), jnp.float32),
            jax.ShapeDtypeStruct((E, H), jnp.float32),
        ),
    )(sa, ra, ea, w1a, w1b, w1c, b1, w2, b2, w3, b3, w4, b4, g, beta)


def _node_mlp(na, agg, w1a, w1b, b1, w2, b2, w3, b3, w4, b4, g, beta, block):
    N = na.shape[0]
    grid = N // block
    row = pl.BlockSpec((block, H), lambda i: (i, 0))
    arow = pl.BlockSpec((2, block, H), lambda i: (0, i, 0))
    wspec = _full((H, H))
    vspec = _full((1, H))
    return pl.pallas_call(
        _node_body,
        grid=(grid,),
        in_specs=[row, arow, wspec, wspec, vspec, wspec, vspec, wspec, vspec,
                  wspec, vspec, vspec, vspec],
        out_specs=row,
        out_shape=jax.ShapeDtypeStruct((N, H), jnp.float32),
    )(na, agg, w1a, w1b, b1, w2, b2, w3, b3, w4, b4, g, beta)


def kernel(node_attr, edge_attr, edge_index, num_nodes,
           eb_W1, eb_b1, eb_W2, eb_b2, eb_W3, eb_b3, eb_W4, eb_b4, eb_g, eb_beta,
           nb_W1, nb_b1, nb_W2, nb_b2, nb_W3, nb_b3, nb_W4, nb_b4, nb_g, nb_beta):
    N = node_attr.shape[0]
    E = edge_attr.shape[0]

    s2d = edge_index[0].reshape(E // CHUNK, 1, CHUNK)
    r2d = edge_index[1].reshape(E // CHUNK, 1, CHUNK)

    sa, ra = _make_gather(N, E)(node_attr, s2d, r2d)

    row = lambda v: v.reshape(1, H)
    y, ye = _edge_mlp(
        sa, ra, edge_attr,
        eb_W1[:H], eb_W1[H:2 * H], eb_W1[2 * H:], row(eb_b1),
        eb_W2, row(eb_b2), eb_W3, row(eb_b3), eb_W4, row(eb_b4),
        row(eb_g), row(eb_beta), block=1000,
    )

    zeros = jnp.zeros((N, H), jnp.float32)
    agg = _make_scatter(N, E)(y, r2d, zeros)

    un = _node_mlp(
        node_attr, agg,
        nb_W1[:H], nb_W1[H:], row(nb_b1),
        nb_W2, row(nb_b2), nb_W3, row(nb_b3), nb_W4, row(nb_b4),
        row(nb_g), row(nb_beta), block=1000,
    )
    return un, ye
